# Initial kernel scaffold; baseline (speedup 1.0000x reference)
import jax
import jax.numpy as jnp
from jax.experimental import pallas as pl


def kernel(x, edge_index, edge_weight, knn_edge_index, knn_edge_weight, W1, b1, W2, b2, scores0, scores1, bias0, bias1, D_k0, D_k1, D_bias0, D_bias1):
    raise NotImplementedError("write your pallas kernel here")


# speedup vs baseline: 3.6800x; 3.6800x over previous
import functools

import jax
import jax.numpy as jnp
from jax import lax
from jax.experimental import pallas as pl
from jax.experimental.pallas import tpu as pltpu
from jax.experimental.pallas import tpu_sc as plsc

_G = 0.1
_CHUNK = 128
_NC = 2
_NS = 16
_NW = _NC * _NS


def _pad_edges(src, dst, w, mult):
    e = src.shape[0]
    ep = -(-e // mult) * mult
    pad = ep - e
    if pad:
        src = jnp.concatenate([src, jnp.zeros((pad,), jnp.int32)])
        dst = jnp.concatenate([dst, jnp.zeros((pad,), jnp.int32)])
        w = jnp.concatenate([w, jnp.zeros((pad,), jnp.float32)])
    return src, dst, w, ep


@functools.lru_cache(maxsize=None)
def _make_spmm(n, d, epa, epk):
    ka = epa // (_NW * _CHUNK)
    kk = epk // (_NW * _CHUNK)
    rps = (n // _NS) // 8 * 8
    tail = n - _NS * rps
    mesh = plsc.VectorSubcoreMesh(core_axis_name="c", subcore_axis_name="s")

    @functools.partial(
        pl.kernel,
        mesh=mesh,
        compiler_params=pltpu.CompilerParams(use_tc_tiling_on_sc=False),
        out_type=(jax.ShapeDtypeStruct((_NC * n, d), jnp.float32),
                  jax.ShapeDtypeStruct((_NC * n, d), jnp.float32)),
        scratch_types=[
            pltpu.VMEM((_CHUNK,), jnp.int32),
            pltpu.VMEM((_CHUNK,), jnp.int32),
            pltpu.VMEM((_CHUNK,), jnp.float32),
            pltpu.VMEM((_CHUNK, d), jnp.float32),
            pltpu.VMEM_SHARED((n, d), jnp.float32),
            pltpu.VMEM_SHARED((n, d), jnp.float32),
            pltpu.SemaphoreType.DMA,
        ],
    )
    def spmm(sup, srcA, dstA, wA, srcK, dstK, wK, zeros,
             outA, outK, src_v, dst_v, w_v, rows_v, accA, accK, sem):
        c = lax.axis_index("c")
        s = lax.axis_index("s")
        wid = c * _NS + s
        lanes = lax.broadcasted_iota(jnp.int32, (16,), 0)

        r0 = pl.multiple_of(s * rps, 8)
        pltpu.sync_copy(zeros.at[pl.ds(0, rps)], accA.at[pl.ds(r0, rps)])
        pltpu.sync_copy(zeros.at[pl.ds(0, rps)], accK.at[pl.ds(r0, rps)])
        if tail:
            @pl.when(s == _NS - 1)
            def _():
                t0 = _NS * rps
                pltpu.sync_copy(zeros.at[pl.ds(0, tail)],
                                accA.at[pl.ds(t0, tail)])
                pltpu.sync_copy(zeros.at[pl.ds(0, tail)],
                                accK.at[pl.ds(t0, tail)])
        plsc.subcore_barrier()

        for srcR, dstR, wR, acc, nch in ((srcA, dstA, wA, accA, ka),
                                         (srcK, dstK, wK, accK, kk)):
            base = wid * (nch * _CHUNK)

            def body(k, carry, srcR=srcR, dstR=dstR, wR=wR, acc=acc,
                     base=base):
                b = pl.multiple_of(base + k * _CHUNK, _CHUNK)
                pltpu.sync_copy(srcR.at[pl.ds(b, _CHUNK)], src_v)
                pltpu.sync_copy(dstR.at[pl.ds(b, _CHUNK)], dst_v)
                pltpu.sync_copy(wR.at[pl.ds(b, _CHUNK)], w_v)
                pltpu.async_copy(sup.at[src_v], rows_v, sem).wait()

                def sbody(e16, c2):
                    e0 = pl.multiple_of(e16 * 16, 16)
                    wvec = w_v[pl.ds(e0, 16)]
                    for lane in range(16):
                        e = e0 + lane
                        w = wvec[lane]
                        for g in range(d // 16):
                            sl = pl.ds(g * 16, 16)
                            rows_v[e, sl] = rows_v[e, sl] * w
                    return c2

                lax.fori_loop(0, _CHUNK // 16, sbody, 0)
                pltpu.sync_copy(rows_v, acc.at[dst_v], add=True)
                return carry

            lax.fori_loop(0, nch, body, 0)

        plsc.subcore_barrier()
        o0 = pl.multiple_of(c * n + r0, 8)
        pltpu.sync_copy(accA.at[pl.ds(r0, rps)], outA.at[pl.ds(o0, rps)])
        pltpu.sync_copy(accK.at[pl.ds(r0, rps)], outK.at[pl.ds(o0, rps)])
        if tail:
            @pl.when(s == _NS - 1)
            def _():
                t0 = _NS * rps
                ot = pl.multiple_of(c * n + t0, 8)
                pltpu.sync_copy(accA.at[pl.ds(t0, tail)],
                                outA.at[pl.ds(ot, tail)])
                pltpu.sync_copy(accK.at[pl.ds(t0, tail)],
                                outK.at[pl.ds(ot, tail)])

    return spmm


@functools.lru_cache(maxsize=None)
def _make_proj(n, f, blk):
    def body(x_ref, w_ref, brow_ref, o_ref):
        o_ref[...] = (jnp.dot(x_ref[...], w_ref[...],
                              preferred_element_type=jnp.float32)
                      + brow_ref[...])

    return pl.pallas_call(
        body,
        grid=(n // blk,),
        in_specs=[
            pl.BlockSpec((blk, f), lambda i: (i, 0)),
            pl.BlockSpec((f, 128), lambda i: (0, 0)),
            pl.BlockSpec((1, 128), lambda i: (0, 0)),
        ],
        out_specs=pl.BlockSpec((blk, 128), lambda i: (i, 0)),
        out_shape=jax.ShapeDtypeStruct((n, 128), jnp.float32),
    )


@functools.lru_cache(maxsize=None)
def _make_comb1(n, nhid, blk):
    def body(p1_ref, hA0, hA1, hK0, hK1, b1row, w2_ref, brow2, o_ref):
        p1 = p1_ref[...]
        sup1 = p1[:, :nhid]
        s = jax.nn.sigmoid(p1[:, nhid:nhid + 1])
        dk = p1[:, nhid + 1:nhid + 2]
        b1 = b1row[...]
        hA = hA0[...] + hA1[...] + b1
        hK = hK0[...] + hK1[...] + b1
        h = s * hA + (1.0 - s) * hK + _G * dk * (sup1 + b1)
        o_ref[...] = (jnp.dot(h, w2_ref[...],
                              preferred_element_type=jnp.float32)
                      + brow2[...])

    part = pl.BlockSpec((blk, nhid), lambda i: (i, 0))
    return pl.pallas_call(
        body,
        grid=(n // blk,),
        in_specs=[
            pl.BlockSpec((blk, 128), lambda i: (i, 0)),
            part, part, part, part,
            pl.BlockSpec((1, nhid), lambda i: (0, 0)),
            pl.BlockSpec((nhid, 128), lambda i: (0, 0)),
            pl.BlockSpec((1, 128), lambda i: (0, 0)),
        ],
        out_specs=pl.BlockSpec((blk, 128), lambda i: (i, 0)),
        out_shape=jax.ShapeDtypeStruct((n, 128), jnp.float32),
    )


@functools.lru_cache(maxsize=None)
def _make_comb2(n, ncls, blk):
    def body(p2_ref, oA0, oA1, oK0, oK1, b2row, o_ref):
        p2 = p2_ref[...]
        sup2 = p2[:, :ncls]
        s = jax.nn.sigmoid(p2[:, ncls:ncls + 1])
        dk = p2[:, ncls + 1:ncls + 2]
        b2 = b2row[...]
        oA = oA0[...] + oA1[...] + b2
        oK = oK0[...] + oK1[...] + b2
        o = s * oA + (1.0 - s) * oK + _G * dk * (sup2 + b2)
        m = jnp.max(o, axis=1, keepdims=True)
        lse = jnp.log(jnp.sum(jnp.exp(o - m), axis=1, keepdims=True)) + m
        o_ref[...] = o - lse

    part = pl.BlockSpec((blk, ncls), lambda i: (i, 0))
    return pl.pallas_call(
        body,
        grid=(n // blk,),
        in_specs=[
            pl.BlockSpec((blk, 128), lambda i: (i, 0)),
            part, part, part, part,
            pl.BlockSpec((1, ncls), lambda i: (0, 0)),
        ],
        out_specs=pl.BlockSpec((blk, ncls), lambda i: (i, 0)),
        out_shape=jax.ShapeDtypeStruct((n, ncls), jnp.float32),
    )


def kernel(x, edge_index, edge_weight, knn_edge_index, knn_edge_weight,
           W1, b1, W2, b2, scores0, scores1, bias0, bias1,
           D_k0, D_k1, D_bias0, D_bias1):
    n, nfeat = x.shape
    nhid = W1.shape[1]
    ncls = W2.shape[1]
    blk = 2000

    mult = _NW * _CHUNK * 2
    sA, dA, wA, epa = _pad_edges(edge_index[1], edge_index[0],
                                 edge_weight, mult)
    sK, dK, wK, epk = _pad_edges(knn_edge_index[1], knn_edge_index[0],
                                 knn_edge_weight, mult)

    wcat1 = (jnp.zeros((nfeat, 128), jnp.float32)
             .at[:, :nhid].set(W1)
             .at[:, nhid].set(scores0[:, 0])
             .at[:, nhid + 1].set(D_k0[:, 0]))
    brow1 = (jnp.zeros((1, 128), jnp.float32)
             .at[0, nhid].set(bias0[0])
             .at[0, nhid + 1].set(D_bias0[0]))
    p1 = _make_proj(n, nfeat, blk)(x, wcat1, brow1)
    sup1 = p1[:, :nhid]

    zrows = max((n // _NS) // 8 * 8, n - _NS * ((n // _NS) // 8 * 8))
    z1 = jnp.zeros((zrows, nhid), jnp.float32)
    hA, hK = _make_spmm(n, nhid, epa, epk)(sup1, sA, dA, wA, sK, dK, wK, z1)

    wcat2 = (jnp.zeros((nhid, 128), jnp.float32)
             .at[:, :ncls].set(W2)
             .at[:, ncls].set(scores1[:, 0])
             .at[:, ncls + 1].set(D_k1[:, 0]))
    brow2 = (jnp.zeros((1, 128), jnp.float32)
             .at[0, ncls].set(bias1[0])
             .at[0, ncls + 1].set(D_bias1[0]))
    p2 = _make_comb1(n, nhid, blk)(p1, hA[:n], hA[n:], hK[:n], hK[n:],
                                   b1[None, :], wcat2, brow2)
    sup2 = p2[:, :ncls]

    z2 = jnp.zeros((zrows, ncls), jnp.float32)
    oA, oK = _make_spmm(n, ncls, epa, epk)(sup2, sA, dA, wA, sK, dK, wK, z2)

    return _make_comb2(n, ncls, blk)(p2, oA[:n], oA[n:], oK[:n], oK[n:],
                                     b2[None, :])
